# Initial kernel scaffold; baseline (speedup 1.0000x reference)
import jax
import jax.numpy as jnp
from jax.experimental import pallas as pl


def kernel(x_nchw, w1, b1, w2, b2, fcw, fcb):
    raise NotImplementedError("write your pallas kernel here")


# speedup vs baseline: 1.6162x; 1.6162x over previous
import numpy as np

import jax
import jax.numpy as jnp
from jax import lax
from jax.experimental import pallas as pl
from jax.experimental.pallas import tpu as pltpu

_PREC = lax.Precision.DEFAULT


def _cnn_kernel(x_ref, a1_ref, b1v_ref, a2_ref, b2v_ref, afc_ref, fcb_ref,
                o_ref, p1_ref, p2_ref):
    f32 = jnp.float32

    a1 = a1_ref[...]
    b1v = b1v_ref[...]
    for py in range(13):
        o = jnp.dot(a1, x_ref[56 * py:56 * py + 112, :],
                    preferred_element_type=f32, precision=_PREC)
        h = jnp.maximum(jnp.maximum(o[0:72], o[72:144]),
                        jnp.maximum(o[144:216], o[216:288]))
        p1_ref[72 * py:72 * py + 72, :] = jnp.maximum(h + b1v, 0.0)

    a2 = a2_ref[...]
    b2v = b2v_ref[...]
    for py2 in range(5):
        base = 144 * py2
        o0 = jnp.dot(a2, p1_ref[base:base + 216, :],
                     preferred_element_type=f32, precision=_PREC)
        o1 = jnp.dot(a2, p1_ref[base + 72:base + 288, :],
                     preferred_element_type=f32, precision=_PREC)
        h = jnp.maximum(jnp.maximum(o0[0:56], o0[56:112]),
                        jnp.maximum(o1[0:56], o1[56:112]))
        p2_ref[56 * py2:56 * py2 + 56, :] = jnp.maximum(h + b2v, 0.0)

    logits = jnp.dot(afc_ref[...], p2_ref[...],
                     preferred_element_type=f32, precision=_PREC)
    logits = logits + fcb_ref[...]
    m = jnp.max(logits, axis=0, keepdims=True)
    s = logits - m
    lse = jnp.log(jnp.sum(jnp.exp(s), axis=0, keepdims=True))
    o_ref[...] = s - lse


def _build_band_matrices(w1, b1, w2, b2, fcw, fcb):
    f32 = jnp.float32

    g, co, px, ky, kx = np.meshgrid(np.arange(4), np.arange(5), np.arange(13),
                                    np.arange(3), np.arange(3), indexing="ij")
    r, p = g // 2, g % 2
    rows = g * 72 + co * 13 + px
    cols = (r + ky) * 28 + (2 * px + p) + kx
    vals = w1.astype(f32).reshape(5, 3, 3)[co, ky, kx]
    a1 = jnp.zeros((288, 112), f32).at[rows, cols].set(vals)

    b1v = jnp.zeros((72, 1), f32).at[
        np.repeat(np.arange(5), 13) * 13 + np.tile(np.arange(13), 5), 0
    ].set(jnp.repeat(b1.astype(f32), 13))

    p2i, co2, px2, ci, ky2, kx2 = np.meshgrid(
        np.arange(2), np.arange(10), np.arange(5), np.arange(5),
        np.arange(3), np.arange(3), indexing="ij")
    rows2 = p2i * 56 + co2 * 5 + px2
    cols2 = ky2 * 72 + ci * 13 + (2 * px2 + p2i) + kx2
    vals2 = w2.astype(f32)[co2, ci, ky2, kx2]
    a2 = jnp.zeros((112, 216), f32).at[rows2, cols2].set(vals2)

    b2v = jnp.zeros((56, 1), f32).at[
        np.repeat(np.arange(10), 5) * 5 + np.tile(np.arange(5), 10), 0
    ].set(jnp.repeat(b2.astype(f32), 5))

    cls, co3, py3, px3 = np.meshgrid(np.arange(10), np.arange(10),
                                     np.arange(5), np.arange(5), indexing="ij")
    afc = jnp.zeros((10, 280), f32).at[
        cls, 56 * py3 + 5 * co3 + px3
    ].set(fcw.astype(f32)[cls, co3 * 25 + py3 * 5 + px3])

    fcb_r = fcb.astype(f32).reshape(10, 1)
    return a1, b1v, a2, b2v, afc, fcb_r


def kernel(x_nchw, w1, b1, w2, b2, fcw, fcb, *, tb=256):
    B = x_nchw.shape[0]
    n_tiles = -(-B // tb)
    b_pad = n_tiles * tb

    x_t = jnp.transpose(x_nchw.astype(jnp.float32).reshape(B, 784), (1, 0))
    if b_pad != B:
        x_t = jnp.pad(x_t, ((0, 0), (0, b_pad - B)))

    a1, b1v, a2, b2v, afc, fcb_r = _build_band_matrices(w1, b1, w2, b2, fcw, fcb)

    out = pl.pallas_call(
        _cnn_kernel,
        out_shape=jax.ShapeDtypeStruct((10, b_pad), jnp.float32),
        grid=(n_tiles,),
        in_specs=[
            pl.BlockSpec((784, tb), lambda i: (0, i)),
            pl.BlockSpec((288, 112), lambda i: (0, 0)),
            pl.BlockSpec((72, 1), lambda i: (0, 0)),
            pl.BlockSpec((112, 216), lambda i: (0, 0)),
            pl.BlockSpec((56, 1), lambda i: (0, 0)),
            pl.BlockSpec((10, 280), lambda i: (0, 0)),
            pl.BlockSpec((10, 1), lambda i: (0, 0)),
        ],
        out_specs=pl.BlockSpec((10, tb), lambda i: (0, i)),
        scratch_shapes=[
            pltpu.VMEM((936, tb), jnp.float32),
            pltpu.VMEM((280, tb), jnp.float32),
        ],
        compiler_params=pltpu.CompilerParams(
            dimension_semantics=("parallel",),
        ),
    )(x_t, a1, b1v, a2, b2v, afc, fcb_r)

    return jnp.transpose(out)[:B]
